# baseline (speedup 1.0000x reference)
import functools

import jax
import jax.numpy as jnp
from jax import lax
from jax.experimental import pallas as pl
from jax.experimental.pallas import tpu as pltpu
from jax.experimental.pallas import tpu_sc as plsc

_NC = 2
_NS = 16
_K = 128
_KB = 2
_NBUF = 2
_DEGW = 16
_CHUNK = _KB * _K


def _sc_aggregate(N, C, EPAD, with_deg):
    CH = C // 2
    T = EPAD // _NS // _CHUNK
    NPAD = -(-(N + 1) // _K) * _K
    TZ = NPAD // _K
    ZPT = -(-TZ // _NS)
    RPT = NPAD // _NS
    assert RPT % 8 == 0 and NPAD % _NS == 0 and T % _NBUF == 0

    mesh = plsc.VectorSubcoreMesh(core_axis_name="c", subcore_axis_name="s",
                                  num_cores=_NC, num_subcores=_NS)

    out_type = [jax.ShapeDtypeStruct((_NC, NPAD, CH), jnp.float32)]
    scratch = (
        [pltpu.VMEM((T, _CHUNK), jnp.int32),
         pltpu.VMEM((T, _CHUNK), jnp.int32)]
        + [pltpu.VMEM((_CHUNK, CH), jnp.float32) for _ in range(_NBUF)]
        + [pltpu.VMEM((_CHUNK, _DEGW), jnp.float32),
           pltpu.VMEM((_K, _DEGW), jnp.float32),
           pltpu.VMEM_SHARED((NPAD, CH), jnp.float32)]
        + [pltpu.SemaphoreType.DMA for _ in range(2 * _NBUF)]
    )
    if with_deg:
        out_type.append(jax.ShapeDtypeStruct((NPAD, _DEGW), jnp.float32))
        scratch += ([pltpu.VMEM_SHARED((NPAD, _DEGW), jnp.float32)]
                    + [pltpu.SemaphoreType.DMA for _ in range(_NBUF)])

    @functools.partial(
        pl.kernel,
        out_type=tuple(out_type),
        mesh=mesh,
        scratch_types=scratch,
        compiler_params=pltpu.CompilerParams(use_tc_tiling_on_sc=False),
    )
    def agg(xs_hbm, cols_hbm, rows_hbm, sum_hbm, *rest):
        if with_deg:
            deg_hbm = rest[0]
            rest = rest[1:]
        idxc_all, idxr_all = rest[0], rest[1]
        rowbufs = rest[2:2 + _NBUF]
        ones_v, zeros16_v, acc_sh = rest[2 + _NBUF:5 + _NBUF]
        gsems = rest[5 + _NBUF:5 + 2 * _NBUF]
        ssems = rest[5 + 2 * _NBUF:5 + 3 * _NBUF]
        if with_deg:
            deg_sh = rest[5 + 3 * _NBUF]
            dsems = rest[6 + 3 * _NBUF:6 + 4 * _NBUF]

        c = lax.axis_index("c")
        s = lax.axis_index("s")

        zeros = jnp.zeros((16,), jnp.float32)
        ones = jnp.ones((16,), jnp.float32)

        def memset_row(i, carry):
            for j in range(CH // 16):
                rowbufs[0][i, pl.ds(16 * j, 16)] = zeros
            ones_v[i, pl.ds(0, 16)] = ones
            return carry
        lax.fori_loop(0, _CHUNK, memset_row, 0)

        def memset_z(i, carry):
            zeros16_v[i, pl.ds(0, 16)] = zeros
            return carry
        lax.fori_loop(0, _K, memset_z, 0)

        def zero_chunk(k, carry):
            g = k * _NS + s
            @pl.when(g < TZ)
            def _():
                r0 = g * _K
                pltpu.sync_copy(rowbufs[0].at[pl.ds(0, _K)],
                                acc_sh.at[pl.ds(r0, _K)])
                if with_deg:
                    pltpu.sync_copy(zeros16_v, deg_sh.at[pl.ds(r0, _K)])
            return carry
        lax.fori_loop(0, ZPT, zero_chunk, 0)

        pltpu.sync_copy(cols_hbm.at[pl.ds(s * T, T)], idxc_all)
        pltpu.sync_copy(rows_hbm.at[pl.ds(s * T, T)], idxr_all)

        plsc.subcore_barrier()

        xc = xs_hbm.at[c]

        def gather(g, b):
            pltpu.async_copy(xc.at[idxc_all.at[g]], rowbufs[b], gsems[b])

        def gather_wait(g, b):
            pltpu.make_async_copy(xc.at[idxc_all.at[g]], rowbufs[b],
                                  gsems[b]).wait()

        def scatter(g, b):
            pltpu.async_copy(rowbufs[b], acc_sh.at[idxr_all.at[g]],
                             ssems[b], add=True)
            if with_deg:
                pltpu.async_copy(ones_v, deg_sh.at[idxr_all.at[g]],
                                 dsems[b], add=True)

        def scatter_wait(g, b):
            pltpu.make_async_copy(rowbufs[b], acc_sh.at[idxr_all.at[g]],
                                  ssems[b]).wait()
            if with_deg:
                pltpu.make_async_copy(ones_v, deg_sh.at[idxr_all.at[g]],
                                      dsems[b]).wait()

        def pipe(u, carry):
            for b in range(_NBUF):
                g = u * _NBUF + b

                @pl.when(g >= _NBUF)
                def _():
                    scatter_wait(g - _NBUF, b)

                gather(g, b)

                gp = g - (_NBUF - 1)
                @pl.when(gp >= 0)
                def _():
                    gather_wait(gp, (b + 1) % _NBUF)
                    scatter(gp, (b + 1) % _NBUF)
            return carry
        lax.fori_loop(0, T // _NBUF, pipe, 0)

        for k in range(_NBUF - 1):
            g = T - (_NBUF - 1) + k
            gather_wait(g, g % _NBUF)
            scatter(g, g % _NBUF)
        for k in range(_NBUF):
            g = T - _NBUF + k
            scatter_wait(g, g % _NBUF)

        plsc.subcore_barrier()

        r0 = s * RPT
        pltpu.sync_copy(acc_sh.at[pl.ds(r0, RPT)], sum_hbm.at[c, pl.ds(r0, RPT)])
        if with_deg:
            @pl.when(c == 0)
            def _():
                pltpu.sync_copy(deg_sh.at[pl.ds(r0, RPT)],
                                deg_hbm.at[pl.ds(r0, RPT)])

    return agg


def _dense_layer(N, C, H, relu, split_out, BN=2000):
    CH = C // 2
    HH = H // 2

    def body(xlo_ref, xhi_ref, plo_ref, phi_ref, deg_ref, w_ref, b_ref, o_ref):
        inv = 1.0 / jnp.maximum(deg_ref[:, 0:1], 1.0)
        terms = ((xlo_ref[0], w_ref[:, :CH]),
                 (xhi_ref[0], w_ref[:, CH:C]),
                 (plo_ref[0] * inv, w_ref[:, C:C + CH]),
                 (phi_ref[0] * inv, w_ref[:, C + CH:]))
        y = b_ref[...]
        for v, w in terms:
            y = y + lax.dot_general(v, w, (((1,), (1,)), ((), ())),
                                    preferred_element_type=jnp.float32)
        if relu:
            y = jnp.maximum(y, 0.0)
        if split_out:
            o_ref[0, :, :] = y[:, :HH]
            o_ref[1, :, :] = y[:, HH:]
        else:
            o_ref[...] = y

    if split_out:
        out_spec = pl.BlockSpec((2, BN, HH), lambda i: (0, i, 0))
        out_shape = jax.ShapeDtypeStruct((2, N, HH), jnp.float32)
    else:
        out_spec = pl.BlockSpec((BN, H), lambda i: (i, 0))
        out_shape = jax.ShapeDtypeStruct((N, H), jnp.float32)

    return pl.pallas_call(
        body,
        grid=(N // BN,),
        in_specs=[
            pl.BlockSpec((1, BN, CH), lambda i: (0, i, 0)),
            pl.BlockSpec((1, BN, CH), lambda i: (1, i, 0)),
            pl.BlockSpec((1, BN, CH), lambda i: (0, i, 0)),
            pl.BlockSpec((1, BN, CH), lambda i: (1, i, 0)),
            pl.BlockSpec((BN, _DEGW), lambda i: (i, 0)),
            pl.BlockSpec((H, 2 * C), lambda i: (0, 0)),
            pl.BlockSpec((1, H), lambda i: (0, 0)),
        ],
        out_specs=out_spec,
        out_shape=out_shape,
    )


def kernel(x, edge_index, W1, b1, W2, b2):
    N, C = x.shape
    H = W1.shape[0]
    O = W2.shape[0]
    E = edge_index.shape[1]
    align = _NS * _CHUNK * _NBUF
    EPAD = -(-E // align) * align

    rows = edge_index[0]
    cols = edge_index[1]
    pad = EPAD - E
    if pad:
        rows = jnp.concatenate([rows, jnp.full((pad,), N, jnp.int32)])
        cols = jnp.concatenate([cols, jnp.zeros((pad,), jnp.int32)])
    rows = rows.reshape(EPAD // _CHUNK, _CHUNK)
    cols = cols.reshape(EPAD // _CHUNK, _CHUNK)

    xs = jnp.stack([x[:, :C // 2], x[:, C // 2:]])
    sums, deg = _sc_aggregate(N, C, EPAD, True)(xs, cols, rows)
    hs = _dense_layer(N, C, H, True, True)(
        xs, xs, sums, sums, deg, W1, b1.reshape(1, H))

    (sums2,) = _sc_aggregate(N, H, EPAD, False)(hs, cols, rows)
    out = _dense_layer(N, H, O, False, False)(
        hs, hs, sums2, sums2, deg, W2, b2.reshape(1, O))
    return out


# speedup vs baseline: 1.3616x; 1.3616x over previous
import functools

import jax
import jax.numpy as jnp
from jax import lax
from jax.experimental import pallas as pl
from jax.experimental.pallas import tpu as pltpu
from jax.experimental.pallas import tpu_sc as plsc

_NC = 2
_NS = 16
_K = 128
_KB = 1
_NBUF = 2
_DEGW = 16
_CHUNK = _KB * _K


def _sc_aggregate(N, C, EPAD, with_deg):
    CH = C // 2
    T = EPAD // _NS // _CHUNK
    NPAD = -(-(N + 1) // _K) * _K
    TZ = NPAD // _K
    ZPT = -(-TZ // _NS)
    RPT = NPAD // _NS
    assert RPT % 8 == 0 and NPAD % _NS == 0 and T % _NBUF == 0

    mesh = plsc.VectorSubcoreMesh(core_axis_name="c", subcore_axis_name="s",
                                  num_cores=_NC, num_subcores=_NS)

    out_type = [jax.ShapeDtypeStruct((_NC, NPAD, CH), jnp.float32)]
    scratch = (
        [pltpu.VMEM((T, _CHUNK), jnp.int32),
         pltpu.VMEM((T, _CHUNK), jnp.int32)]
        + [pltpu.VMEM((_CHUNK, CH), jnp.float32) for _ in range(_NBUF)]
        + [pltpu.VMEM((_CHUNK, _DEGW), jnp.float32),
           pltpu.VMEM((_K, _DEGW), jnp.float32),
           pltpu.VMEM_SHARED((NPAD, CH), jnp.float32)]
        + [pltpu.SemaphoreType.DMA for _ in range(2 * _NBUF)]
    )
    if with_deg:
        out_type.append(jax.ShapeDtypeStruct((NPAD, _DEGW), jnp.float32))
        scratch += ([pltpu.VMEM_SHARED((NPAD, _DEGW), jnp.float32)]
                    + [pltpu.SemaphoreType.DMA for _ in range(_NBUF)])

    @functools.partial(
        pl.kernel,
        out_type=tuple(out_type),
        mesh=mesh,
        scratch_types=scratch,
        compiler_params=pltpu.CompilerParams(use_tc_tiling_on_sc=False),
    )
    def agg(xs_hbm, cols_hbm, rows_hbm, sum_hbm, *rest):
        if with_deg:
            deg_hbm = rest[0]
            rest = rest[1:]
        idxc_all, idxr_all = rest[0], rest[1]
        rowbufs = rest[2:2 + _NBUF]
        ones_v, zeros16_v, acc_sh = rest[2 + _NBUF:5 + _NBUF]
        gsems = rest[5 + _NBUF:5 + 2 * _NBUF]
        ssems = rest[5 + 2 * _NBUF:5 + 3 * _NBUF]
        if with_deg:
            deg_sh = rest[5 + 3 * _NBUF]
            dsems = rest[6 + 3 * _NBUF:6 + 4 * _NBUF]

        c = lax.axis_index("c")
        s = lax.axis_index("s")

        zeros = jnp.zeros((16,), jnp.float32)
        ones = jnp.ones((16,), jnp.float32)

        def memset_row(i, carry):
            for j in range(CH // 16):
                rowbufs[0][i, pl.ds(16 * j, 16)] = zeros
            ones_v[i, pl.ds(0, 16)] = ones
            return carry
        lax.fori_loop(0, _CHUNK, memset_row, 0)

        def memset_z(i, carry):
            zeros16_v[i, pl.ds(0, 16)] = zeros
            return carry
        lax.fori_loop(0, _K, memset_z, 0)

        def zero_chunk(k, carry):
            g = k * _NS + s
            @pl.when(g < TZ)
            def _():
                r0 = g * _K
                pltpu.sync_copy(rowbufs[0].at[pl.ds(0, _K)],
                                acc_sh.at[pl.ds(r0, _K)])
                if with_deg:
                    pltpu.sync_copy(zeros16_v, deg_sh.at[pl.ds(r0, _K)])
            return carry
        lax.fori_loop(0, ZPT, zero_chunk, 0)

        pltpu.sync_copy(cols_hbm.at[pl.ds(s * T, T)], idxc_all)
        pltpu.sync_copy(rows_hbm.at[pl.ds(s * T, T)], idxr_all)

        plsc.subcore_barrier()

        xc = xs_hbm.at[c]

        def gather(g, b):
            pltpu.async_copy(xc.at[idxc_all.at[g]], rowbufs[b], gsems[b])

        def gather_wait(g, b):
            pltpu.make_async_copy(xc.at[idxc_all.at[g]], rowbufs[b],
                                  gsems[b]).wait()

        def scatter(g, b):
            pltpu.async_copy(rowbufs[b], acc_sh.at[idxr_all.at[g]],
                             ssems[b], add=True)
            if with_deg:
                pltpu.async_copy(ones_v, deg_sh.at[idxr_all.at[g]],
                                 dsems[b], add=True)

        def scatter_wait(g, b):
            pltpu.make_async_copy(rowbufs[b], acc_sh.at[idxr_all.at[g]],
                                  ssems[b]).wait()
            if with_deg:
                pltpu.make_async_copy(ones_v, deg_sh.at[idxr_all.at[g]],
                                      dsems[b]).wait()

        def pipe(u, carry):
            for b in range(_NBUF):
                g = u * _NBUF + b

                @pl.when(g >= _NBUF)
                def _():
                    scatter_wait(g - _NBUF, b)

                gather(g, b)

                gp = g - (_NBUF - 1)
                @pl.when(gp >= 0)
                def _():
                    gather_wait(gp, (b + 1) % _NBUF)
                    scatter(gp, (b + 1) % _NBUF)
            return carry
        lax.fori_loop(0, T // _NBUF, pipe, 0)

        for k in range(_NBUF - 1):
            g = T - (_NBUF - 1) + k
            gather_wait(g, g % _NBUF)
            scatter(g, g % _NBUF)
        for k in range(_NBUF):
            g = T - _NBUF + k
            scatter_wait(g, g % _NBUF)

        plsc.subcore_barrier()

        r0 = s * RPT
        pltpu.sync_copy(acc_sh.at[pl.ds(r0, RPT)], sum_hbm.at[c, pl.ds(r0, RPT)])
        if with_deg:
            @pl.when(c == 0)
            def _():
                pltpu.sync_copy(deg_sh.at[pl.ds(r0, RPT)],
                                deg_hbm.at[pl.ds(r0, RPT)])

    return agg


def _dense_layer(N, C, H, relu, split_out, BN=2000):
    CH = C // 2
    HH = H // 2

    def body(xlo_ref, xhi_ref, plo_ref, phi_ref, deg_ref, w_ref, b_ref, o_ref):
        inv = 1.0 / jnp.maximum(deg_ref[:, 0:1], 1.0)
        terms = ((xlo_ref[0], w_ref[:, :CH]),
                 (xhi_ref[0], w_ref[:, CH:C]),
                 (plo_ref[0] * inv, w_ref[:, C:C + CH]),
                 (phi_ref[0] * inv, w_ref[:, C + CH:]))
        y = b_ref[...]
        for v, w in terms:
            y = y + lax.dot_general(v, w, (((1,), (1,)), ((), ())),
                                    preferred_element_type=jnp.float32)
        if relu:
            y = jnp.maximum(y, 0.0)
        if split_out:
            o_ref[0, :, :] = y[:, :HH]
            o_ref[1, :, :] = y[:, HH:]
        else:
            o_ref[...] = y

    if split_out:
        out_spec = pl.BlockSpec((2, BN, HH), lambda i: (0, i, 0))
        out_shape = jax.ShapeDtypeStruct((2, N, HH), jnp.float32)
    else:
        out_spec = pl.BlockSpec((BN, H), lambda i: (i, 0))
        out_shape = jax.ShapeDtypeStruct((N, H), jnp.float32)

    return pl.pallas_call(
        body,
        grid=(N // BN,),
        in_specs=[
            pl.BlockSpec((1, BN, CH), lambda i: (0, i, 0)),
            pl.BlockSpec((1, BN, CH), lambda i: (1, i, 0)),
            pl.BlockSpec((1, BN, CH), lambda i: (0, i, 0)),
            pl.BlockSpec((1, BN, CH), lambda i: (1, i, 0)),
            pl.BlockSpec((BN, _DEGW), lambda i: (i, 0)),
            pl.BlockSpec((H, 2 * C), lambda i: (0, 0)),
            pl.BlockSpec((1, H), lambda i: (0, 0)),
        ],
        out_specs=out_spec,
        out_shape=out_shape,
    )


def kernel(x, edge_index, W1, b1, W2, b2):
    N, C = x.shape
    H = W1.shape[0]
    O = W2.shape[0]
    E = edge_index.shape[1]
    align = _NS * _CHUNK * _NBUF
    EPAD = -(-E // align) * align

    rows = edge_index[0]
    cols = edge_index[1]
    pad = EPAD - E
    if pad:
        rows = jnp.concatenate([rows, jnp.full((pad,), N, jnp.int32)])
        cols = jnp.concatenate([cols, jnp.zeros((pad,), jnp.int32)])
    rows = rows.reshape(EPAD // _CHUNK, _CHUNK)
    cols = cols.reshape(EPAD // _CHUNK, _CHUNK)

    xs = jnp.stack([x[:, :C // 2], x[:, C // 2:]])
    sums, deg = _sc_aggregate(N, C, EPAD, True)(xs, cols, rows)
    hs = _dense_layer(N, C, H, True, True)(
        xs, xs, sums, sums, deg, W1, b1.reshape(1, H))

    (sums2,) = _sc_aggregate(N, H, EPAD, False)(hs, cols, rows)
    out = _dense_layer(N, H, O, False, False)(
        hs, hs, sums2, sums2, deg, W2, b2.reshape(1, O))
    return out
